# baseline (speedup 1.0000x reference)
import jax
import jax.numpy as jnp
from jax import lax
from jax.experimental import pallas as pl
from jax.experimental.pallas import tpu as pltpu
from jax.experimental.pallas import tpu_sc as plsc

B = 16384
NN = 13
NCAT = 26
CARD = 100000
D = 32
NTOK = 1 + NN + NCAT

NC = 2
NS = 16
NW = NC * NS
BW = B // NW
CB = 32
NCHUNK = BW // CB


def _bc(x):
    return jnp.broadcast_to(x, (16,))


def _sc_body(numT, catT, w_hbm, bias_hbm, cls_hbm, tables, out,
             craw, nraw, gbuf, stage, w_v, bias_v, cls_v, gsem, wsem):
    wid = lax.axis_index("s") * NC + lax.axis_index("c")
    base = pl.multiple_of(wid * BW, BW)

    pltpu.sync_copy(w_hbm, w_v)
    pltpu.sync_copy(bias_hbm, bias_v)
    pltpu.sync_copy(cls_hbm, cls_v)

    iota = lax.iota(jnp.int32, 16)
    rows0 = iota * D
    rows1 = (iota + 16) * D
    bidx0 = iota
    bidx1 = iota + 16

    def chunk(t, carry):
        b0 = pl.multiple_of(base + t * CB, CB)
        pltpu.sync_copy(catT.at[:, pl.ds(b0, CB)], craw)
        pltpu.sync_copy(numT.at[:, pl.ds(b0, CB)], nraw)

        gh = []
        for i in range(NCAT):
            for k in range(CB // 16):
                craw[i, pl.ds(16 * k, 16)] = craw[i, pl.ds(16 * k, 16)] + (i * CARD)
            gh.append(pltpu.async_copy(tables.at[craw.at[i]], gbuf.at[i], gsem))

        def cls_loop(d, c):
            cv = plsc.load_gather(cls_v, [_bc(d)])
            stage[d, pl.ds(0, 16)] = cv
            stage[d, pl.ds(16, 16)] = cv
            return c
        lax.fori_loop(0, D, cls_loop, 0)

        def jloop(j, c):
            v0 = nraw[j, pl.ds(0, 16)]
            v1 = nraw[j, pl.ds(16, 16)]

            def dloop(d, c2):
                bw = plsc.load_gather(w_v, [_bc(j), _bc(d)])
                bb = plsc.load_gather(bias_v, [_bc(j), _bc(d)])
                r = (1 + j) * D + d
                stage[r, pl.ds(0, 16)] = v0 * bw + bb
                stage[r, pl.ds(16, 16)] = v1 * bw + bb
                return c2
            lax.fori_loop(0, D, dloop, 0)
            return c
        lax.fori_loop(0, NN, jloop, 0)

        for h in gh:
            h.wait()

        def iloop(i, c):
            bci = _bc(i)

            def dloop2(d, c2):
                bcd = _bc(d)
                col0 = plsc.load_gather(gbuf, [bci, bidx0, bcd])
                col1 = plsc.load_gather(gbuf, [bci, bidx1, bcd])
                r = (1 + NN + i) * D + d
                stage[r, pl.ds(0, 16)] = col0
                stage[r, pl.ds(16, 16)] = col1
                return c2
            lax.fori_loop(0, D, dloop2, 0)
            return c
        lax.fori_loop(0, NCAT, iloop, 0)

        pltpu.async_copy(stage, out.at[:, pl.ds(b0, CB)], wsem).wait()
        return carry
    lax.fori_loop(0, NCHUNK, chunk, 0)


def kernel(numeric, categorical, num_weight, num_bias, cat_tables, cls_token):
    numT = numeric.T
    catT = categorical.T
    tables = cat_tables.reshape(NCAT * CARD, D)
    cls = cls_token.reshape(D)
    mesh = plsc.VectorSubcoreMesh(core_axis_name="c", subcore_axis_name="s")
    fn = pl.kernel(
        _sc_body,
        out_type=jax.ShapeDtypeStruct((NTOK * D, B), jnp.float32),
        mesh=mesh,
        scratch_types=[
            pltpu.VMEM((NCAT, CB), jnp.int32),
            pltpu.VMEM((NN, CB), jnp.float32),
            pltpu.VMEM((NCAT, CB, D), jnp.float32),
            pltpu.VMEM((NTOK * D, CB), jnp.float32),
            pltpu.VMEM((NN, D), jnp.float32),
            pltpu.VMEM((NN, D), jnp.float32),
            pltpu.VMEM((D,), jnp.float32),
            pltpu.SemaphoreType.DMA,
            pltpu.SemaphoreType.DMA,
        ],
        compiler_params=pltpu.CompilerParams(use_tc_tiling_on_sc=False,
                                             needs_layout_passes=False),
    )
    out = fn(numT, catT, num_weight, num_bias, cls, tables)
    return out.reshape(NTOK, D, B).transpose(2, 0, 1)


# speedup vs baseline: 1.0327x; 1.0327x over previous
import jax
import jax.numpy as jnp
from jax import lax
from jax.experimental import pallas as pl
from jax.experimental.pallas import tpu as pltpu
from jax.experimental.pallas import tpu_sc as plsc

B = 16384
NN = 13
NCAT = 26
CARD = 100000
D = 32
NTOK = 1 + NN + NCAT
GPR = 128 // D

NC = 2
NS = 16
NW = NC * NS
BW = B // NW
SLAB = 128
NSLAB = BW // SLAB
NGB = 4


def _bc(x):
    return jnp.broadcast_to(x, (16,))


def _sc_body(numT, catT, w_hbm, bias_hbm, cls_hbm, tables, out,
             craw, soff, nraw, gbuf, stg, w_v, bias_v, cls_v, gsem, wsem):
    wid = lax.axis_index("s") * NC + lax.axis_index("c")
    base = pl.multiple_of(wid * BW, BW)

    pltpu.sync_copy(w_hbm, w_v)
    pltpu.sync_copy(bias_hbm, bias_v)
    pltpu.sync_copy(cls_hbm, cls_v)

    iota = lax.iota(jnp.int32, 16)
    bidx = [iota + 16 * k for k in range(SLAB // 16)]

    def slab(s, carry):
        b0 = pl.multiple_of(base + s * SLAB, SLAB)
        pltpu.sync_copy(catT.at[:, pl.ds(b0, SLAB)], craw)
        pltpu.sync_copy(numT.at[:, pl.ds(b0, SLAB)], nraw)

        for i in range(NCAT):
            for k in range(SLAB // 16):
                c = craw[i, pl.ds(16 * k, 16)]
                soff[i, pl.ds(16 * k, 16)] = (c & (GPR - 1)) * D
                craw[i, pl.ds(16 * k, 16)] = (c + i * CARD) >> 2
        gh = [None] * NCAT

        def fire(i):
            gh[i] = pltpu.async_copy(
                tables.at[craw.at[i]], gbuf.at[i % NGB], gsem)
        for i in range(NGB):
            fire(i)

        wh = [None, None]

        def emit(t, fill):
            st = stg.at[t % 2]
            if wh[t % 2] is not None:
                wh[t % 2].wait()
            fill(st)
            wh[t % 2] = pltpu.async_copy(
                st, out.at[pl.ds(t * D, D), pl.ds(b0, SLAB)], wsem)

        def fill_cls(st):
            def dloop(d, c):
                cv = plsc.load_gather(cls_v, [_bc(d)])
                for k in range(SLAB // 16):
                    st[d, pl.ds(16 * k, 16)] = cv
                return c
            lax.fori_loop(0, D, dloop, 0)
        emit(0, fill_cls)

        for j in range(NN):
            def fill_num(st, j=j):
                def dloop(d, c):
                    bw = plsc.load_gather(w_v, [_bc(j), _bc(d)])
                    bb = plsc.load_gather(bias_v, [_bc(j), _bc(d)])
                    for k in range(SLAB // 16):
                        st[d, pl.ds(16 * k, 16)] = nraw[j, pl.ds(16 * k, 16)] * bw + bb
                    return c
                lax.fori_loop(0, D, dloop, 0)
            emit(1 + j, fill_num)

        for i in range(NCAT):
            gh[i].wait()

            def fill_cat(st, i=i):
                g = gbuf.at[i % NGB]
                for k in range(SLAB // 16):
                    sk = soff[i, pl.ds(16 * k, 16)]

                    def dloop(d, c, k=k, sk=sk):
                        st[d, pl.ds(16 * k, 16)] = plsc.load_gather(
                            g, [bidx[k], sk + d])
                        return c
                    lax.fori_loop(0, D, dloop, 0)
            emit(1 + NN + i, fill_cat)
            if i + NGB < NCAT:
                fire(i + NGB)

        wh[0].wait()
        wh[1].wait()
        return carry
    lax.fori_loop(0, NSLAB, slab, 0)


def kernel(numeric, categorical, num_weight, num_bias, cat_tables, cls_token):
    numT = numeric.T
    catT = categorical.T
    tables = cat_tables.reshape(NCAT * CARD // GPR, GPR * D)
    cls = cls_token.reshape(D)
    mesh = plsc.VectorSubcoreMesh(core_axis_name="c", subcore_axis_name="s")
    fn = pl.kernel(
        _sc_body,
        out_type=jax.ShapeDtypeStruct((NTOK * D, B), jnp.float32),
        mesh=mesh,
        scratch_types=[
            pltpu.VMEM((NCAT, SLAB), jnp.int32),
            pltpu.VMEM((NCAT, SLAB), jnp.int32),
            pltpu.VMEM((NN, SLAB), jnp.float32),
            pltpu.VMEM((NGB, SLAB, GPR * D), jnp.float32),
            pltpu.VMEM((2, D, SLAB), jnp.float32),
            pltpu.VMEM((NN, D), jnp.float32),
            pltpu.VMEM((NN, D), jnp.float32),
            pltpu.VMEM((D,), jnp.float32),
            pltpu.SemaphoreType.DMA,
            pltpu.SemaphoreType.DMA,
        ],
        compiler_params=pltpu.CompilerParams(use_tc_tiling_on_sc=True,
                                             needs_layout_passes=False),
    )
    out = fn(numT, catT, num_weight, num_bias, cls, tables)
    return out.reshape(NTOK, D, B).transpose(2, 0, 1)


# speedup vs baseline: 1.4602x; 1.4140x over previous
import jax
import jax.numpy as jnp
from jax import lax
from jax.experimental import pallas as pl
from jax.experimental.pallas import tpu as pltpu
from jax.experimental.pallas import tpu_sc as plsc

B = 16384
NN = 13
NCAT = 26
CARD = 100000
D = 32
NTOK = 1 + NN + NCAT

NC = 2
NS = 16
NW = NC * NS
BW = B // NW
SLAB = 128
NSLAB = BW // SLAB
NSTG = 6


def _bc(x):
    return jnp.broadcast_to(x, (16,))


def _sc_body(numT, catT, w_hbm, bias_hbm, cls_hbm, tabT, out,
             craw, nraw, stg, w_v, bias_v, cls_v,
             gs0, gs1, gs2, gs3, wsem):
    gsems = [gs0, gs1, gs2, gs3]
    wid = lax.axis_index("s") * NC + lax.axis_index("c")
    base = pl.multiple_of(wid * BW, BW)

    pltpu.sync_copy(w_hbm, w_v)
    pltpu.sync_copy(bias_hbm, bias_v)
    pltpu.sync_copy(cls_hbm, cls_v)

    def slab(s, carry):
        b0 = pl.multiple_of(base + s * SLAB, SLAB)
        pltpu.sync_copy(catT.at[:, pl.ds(b0, SLAB)], craw)
        pltpu.sync_copy(numT.at[:, pl.ds(b0, SLAB)], nraw)

        wh = [None] * NSTG

        def prefire(i):
            slot = 2 + (i % 4)
            if wh[slot] is not None:
                wh[slot].wait()
                wh[slot] = None
            idx = craw.at[i]
            sem = gsems[i % 4]

            def dfire(d, c):
                pltpu.async_copy(tabT.at[i, d].at[idx], stg.at[slot, d], sem)
                return c
            lax.fori_loop(0, D, dfire, 0)

        def emit(t, slot):
            wh[slot] = pltpu.async_copy(
                stg.at[slot], out.at[pl.ds(t * D, D), pl.ds(b0, SLAB)], wsem)

        for i in range(3):
            prefire(i)

        def fill_cls(st):
            def dloop(d, c):
                cv = plsc.load_gather(cls_v, [_bc(d)])
                for k in range(SLAB // 16):
                    st[d, pl.ds(16 * k, 16)] = cv
                return c
            lax.fori_loop(0, D, dloop, 0)
        fill_cls(stg.at[0])
        emit(0, 0)

        for j in range(NN):
            slot = (j + 1) % 2
            if wh[slot] is not None:
                wh[slot].wait()

            def fill_num(st, j=j):
                def dloop(d, c):
                    bw = plsc.load_gather(w_v, [_bc(j), _bc(d)])
                    bb = plsc.load_gather(bias_v, [_bc(j), _bc(d)])
                    for k in range(SLAB // 16):
                        st[d, pl.ds(16 * k, 16)] = nraw[j, pl.ds(16 * k, 16)] * bw + bb
                    return c
                lax.fori_loop(0, D, dloop, 0)
            fill_num(stg.at[slot])
            emit(1 + j, slot)

        for i in range(NCAT):
            slot = 2 + (i % 4)
            pltpu.make_async_copy(
                tabT.at[0].at[:, pl.ds(0, SLAB)], stg.at[slot],
                gsems[i % 4]).wait()
            emit(1 + NN + i, slot)
            if i + 3 < NCAT:
                prefire(i + 3)

        for h in wh:
            if h is not None:
                h.wait()
        return carry
    lax.fori_loop(0, NSLAB, slab, 0)


def kernel(numeric, categorical, num_weight, num_bias, cat_tables, cls_token):
    numT = numeric.T
    catT = categorical.T
    tabT = cat_tables.transpose(0, 2, 1)
    cls = cls_token.reshape(D)
    mesh = plsc.VectorSubcoreMesh(core_axis_name="c", subcore_axis_name="s")
    fn = pl.kernel(
        _sc_body,
        out_type=jax.ShapeDtypeStruct((NTOK * D, B), jnp.float32),
        mesh=mesh,
        scratch_types=[
            pltpu.VMEM((NCAT, SLAB), jnp.int32),
            pltpu.VMEM((NN, SLAB), jnp.float32),
            pltpu.VMEM((NSTG, D, SLAB), jnp.float32),
            pltpu.VMEM((NN, D), jnp.float32),
            pltpu.VMEM((NN, D), jnp.float32),
            pltpu.VMEM((D,), jnp.float32),
            pltpu.SemaphoreType.DMA,
            pltpu.SemaphoreType.DMA,
            pltpu.SemaphoreType.DMA,
            pltpu.SemaphoreType.DMA,
            pltpu.SemaphoreType.DMA,
        ],
        compiler_params=pltpu.CompilerParams(use_tc_tiling_on_sc=False,
                                             needs_layout_passes=False),
    )
    out = fn(numT, catT, num_weight, num_bias, cls, tabT)
    return out.reshape(NTOK, D, B).transpose(2, 0, 1)
